# Initial kernel scaffold; baseline (speedup 1.0000x reference)
import jax
import jax.numpy as jnp
from jax.experimental import pallas as pl


def kernel(x, edge_index, batch, W1, as1, ad1, b1, W2, as2, ad2, b2, W3, as3, ad3, b3, Wl, bl, Wc, bc):
    raise NotImplementedError("write your pallas kernel here")


# speedup vs baseline: 9.9494x; 9.9494x over previous
import functools

import jax
import jax.numpy as jnp
from jax import lax
from jax.experimental import pallas as pl
from jax.experimental.pallas import tpu as pltpu
from jax.experimental.pallas import tpu_sc as plsc

N = 10000
NP = 10240
E = 160000
D = 512
BLK = 1024
NC, NS = 2, 16
RPT = NP // NS
B = 128
NBT = E // B
NB_HALF = NBT // (NC * NS)
LEFT = NBT // NC - NS * NB_HALF
AC_W = 128
GB = 32

_f32 = jnp.float32


def _dot(a, b):
    return jnp.dot(a, b, preferred_element_type=_f32)


def _tc_first_body(x_ref, w_ref, a_ref,
                   h0, h1, h2, h3, ac_ref):
    h = _dot(x_ref[...], w_ref[...])
    ac_ref[...] = jnp.pad(_dot(h, a_ref[...]), ((0, 0), (0, AC_W - 32)))
    for k, hr in enumerate((h0, h1, h2, h3)):
        hr[...] = h[:, 128 * k:128 * (k + 1)]


def _tc_mid_body(p_ref, b_ref, w_ref, a_ref,
                 h0, h1, h2, h3, ac_ref):
    p = p_ref[...]
    hin = jnp.concatenate([p[0, k] + p[1, k] for k in range(4)], axis=-1)
    hin = jnp.maximum(hin + b_ref[...], 0.0)
    h = _dot(hin, w_ref[...])
    ac_ref[...] = jnp.pad(_dot(h, a_ref[...]), ((0, 0), (0, AC_W - 32)))
    for k, hr in enumerate((h0, h1, h2, h3)):
        hr[...] = h[:, 128 * k:128 * (k + 1)]


def _tc_final_body(p_ref, b3_ref, wl_ref, bl_ref, wc_ref, bc_ref, out_ref):
    p = p_ref[...]
    hin = jnp.concatenate([p[0, k] + p[1, k] for k in range(4)], axis=-1)
    hin = jnp.maximum(hin + b3_ref[...], 0.0)
    hl = jnp.maximum(_dot(hin, wl_ref[...]) + bl_ref[...], 0.0)
    out_ref[...] = _dot(hl, wc_ref[...]) + bc_ref[...]


def _const_spec(shape):
    return pl.BlockSpec(shape, lambda b: (0,) * len(shape))


def _row_spec(shape):
    return pl.BlockSpec(shape, lambda b: (b,) + (0,) * (len(shape) - 1))


_H_OUT = (
    [jax.ShapeDtypeStruct((NP, 128), _f32) for _ in range(4)]
    + [jax.ShapeDtypeStruct((NP, AC_W), _f32)]
)
_H_OUT_SPECS = (
    [_row_spec((BLK, 128)) for _ in range(4)]
    + [_row_spec((BLK, AC_W))]
)


@jax.jit
def _tc_first(x_p, w, a):
    return pl.pallas_call(
        _tc_first_body,
        grid=(NP // BLK,),
        in_specs=[
            _row_spec((BLK, 128)),
            _const_spec((128, D)),
            _const_spec((D, 32)),
        ],
        out_specs=_H_OUT_SPECS,
        out_shape=_H_OUT,
    )(x_p, w, a)


@jax.jit
def _tc_mid(p, bprev, w, a):
    return pl.pallas_call(
        _tc_mid_body,
        grid=(NP // BLK,),
        in_specs=[
            pl.BlockSpec((2, 4, BLK, 128), lambda b: (0, 0, b, 0)),
            _const_spec((1, D)),
            _const_spec((D, D)),
            _const_spec((D, 32)),
        ],
        out_specs=_H_OUT_SPECS,
        out_shape=_H_OUT,
    )(p, bprev, w, a)


@jax.jit
def _tc_final(p, b3, wl, bl, wc, bc):
    return pl.pallas_call(
        _tc_final_body,
        grid=(NP // BLK,),
        in_specs=[
            pl.BlockSpec((2, 4, BLK, 128), lambda b: (0, 0, b, 0)),
            _const_spec((1, D)),
            _const_spec((D, D)),
            _const_spec((1, D)),
            _const_spec((D, 3)),
            _const_spec((1, 3)),
        ],
        out_specs=_row_spec((BLK, 3)),
        out_shape=jax.ShapeDtypeStruct((NP, 3), _f32),
    )(p, b3, wl, bl, wc, bc)


def _sc_body(ei, h0, h1, h2, h3, ac,
             p_out, a_out,
             acc_sh,
             eidx,
             gs, gd, exw, hrow, alph_v):
    c = lax.axis_index("c")
    s = lax.axis_index("s")
    rows0 = s * RPT

    zero16 = jnp.zeros((16,), _f32)

    def _zero_rows(ref):
        def zb(i, _):
            for v in range(8):
                ref[i, pl.ds(16 * v, 16)] = zero16
            return 0
        lax.fori_loop(0, B, zb, 0)

    def _zero_acc():
        for j in range(RPT // B):
            pltpu.sync_copy(hrow, acc_sh.at[pl.ds(rows0 + j * B, B)])

    _zero_rows(exw)
    _zero_rows(hrow)
    _zero_acc()
    plsc.subcore_barrier()

    HALF = NBT // NC
    base_own = c * HALF + s * NB_HALF
    base_mir = (1 - c) * HALF + s * NB_HALF
    nb = NB_HALF + jnp.where(s == NS - 1, LEFT, 0)

    def _eoff(batch_idx):
        return pl.multiple_of(batch_idx * B, B)

    def _gather_ex():
        for q in range(B // GB):
            pltpu.sync_copy(ac.at[eidx.at[0, pl.ds(GB * q, GB)]], gs)
            pltpu.sync_copy(ac.at[eidx.at[1, pl.ds(GB * q, GB)]], gd)

            def body(i, _):
                e = gs[i, pl.ds(0, 16)] + gd[i, pl.ds(16, 16)]
                e = jnp.maximum(e, 0.2 * e)
                exw[GB * q + i, pl.ds(0, 16)] = jnp.exp(e)
                return 0

            lax.fori_loop(0, GB, body, 0)

    def _a1(bi, store):
        off = _eoff(bi)
        pltpu.sync_copy(ei.at[:, pl.ds(off, B)], eidx)
        _gather_ex()
        if store:
            def st(i8, _2):
                for v in range(8):
                    alph_v[i8, pl.ds(16 * v, 16)] = \
                        exw[8 * i8 + v, pl.ds(0, 16)]
                return 0

            lax.fori_loop(0, B // 8, st, 0)
            pltpu.sync_copy(alph_v, a_out.at[bi])
        pltpu.sync_copy(exw, acc_sh.at[eidx.at[1]], add=True)
        return 0

    lax.fori_loop(base_own, base_own + nb,
                  lambda bi, x: _a1(bi, True), 0)
    lax.fori_loop(base_mir, base_mir + nb,
                  lambda bi, x: _a1(bi, False), 0)
    plsc.subcore_barrier()

    def _a2(bi, _):
        off = _eoff(bi)
        pltpu.sync_copy(ei.at[:, pl.ds(off, B)], eidx)
        pltpu.sync_copy(acc_sh.at[eidx.at[1]], hrow)
        pltpu.sync_copy(a_out.at[bi], alph_v)

        def body(i8, _2):
            for v in range(8):
                alph_v[i8, pl.ds(16 * v, 16)] = (
                    alph_v[i8, pl.ds(16 * v, 16)]
                    / (hrow[8 * i8 + v, pl.ds(0, 16)] + 1e-16))
            return 0

        lax.fori_loop(0, B // 8, body, 0)
        pltpu.sync_copy(alph_v, a_out.at[bi])
        return 0

    lax.fori_loop(base_own, base_own + nb, _a2, 0)
    plsc.subcore_barrier()

    _zero_rows(hrow)
    _zero_acc()
    plsc.subcore_barrier()

    for k, hk in enumerate((h0, h1, h2, h3)):
        def _b(bi, _, k=k, hk=hk):
            off = _eoff(bi)
            pltpu.sync_copy(ei.at[:, pl.ds(off, B)], eidx)
            pltpu.sync_copy(hk.at[eidx.at[0]], hrow)
            pltpu.sync_copy(a_out.at[bi], alph_v)

            def body(i8, _2):
                for v in range(8):
                    arow = alph_v[i8, pl.ds(16 * v, 16)]
                    a0 = arow[2 * k]
                    a1 = arow[2 * k + 1]
                    r = 8 * i8 + v
                    for u in range(8):
                        au = a0 if u < 4 else a1
                        hrow[r, pl.ds(16 * u, 16)] = \
                            hrow[r, pl.ds(16 * u, 16)] * au
                return 0

            lax.fori_loop(0, B // 8, body, 0)
            pltpu.sync_copy(hrow, acc_sh.at[eidx.at[1]], add=True)
            return 0

        lax.fori_loop(base_own, base_own + nb, _b, 0)
        plsc.subcore_barrier()

        @pl.when(c == 0)
        def _():
            pltpu.sync_copy(acc_sh.at[pl.ds(rows0, RPT)],
                            p_out.at[0, k, pl.ds(rows0, RPT)])

        @pl.when(c == 1)
        def _():
            pltpu.sync_copy(acc_sh.at[pl.ds(rows0, RPT)],
                            p_out.at[1, k, pl.ds(rows0, RPT)])

        if k < 3:
            _zero_rows(hrow)
            _zero_acc()
        plsc.subcore_barrier()


@functools.cache
def _sc_layer_fn():
    mesh = plsc.VectorSubcoreMesh(
        core_axis_name="c", subcore_axis_name="s",
        num_cores=NC, num_subcores=NS)
    return pl.kernel(
    _sc_body,
        out_type=[jax.ShapeDtypeStruct((2, 4, NP, 128), _f32),
                  jax.ShapeDtypeStruct((NBT, B // 8, 128), _f32)],
        mesh=mesh,
        scratch_types=[
            pltpu.VMEM_SHARED((NP, 128), _f32),
            pltpu.VMEM((2, B), jnp.int32),
            pltpu.VMEM((GB, AC_W), _f32),
            pltpu.VMEM((GB, AC_W), _f32),
            pltpu.VMEM((B, 128), _f32),
            pltpu.VMEM((B, 128), _f32),
            pltpu.VMEM((B // 8, 128), _f32),
        ],
    )


def _att_mat(a):
    h, cdim = a.shape
    if h == 1:
        m = jnp.tile(a.reshape(-1, 1), (1, 8))
    else:
        m = (a[:, :, None] * jnp.eye(h, dtype=a.dtype)[:, None, :]).reshape(h * cdim, h)
    return jnp.concatenate([m, m], axis=1)


def _att_pair(a_s, a_d):
    return jnp.concatenate([_att_mat(a_s), _att_mat(a_d)], axis=1)


def kernel(x, edge_index, batch, W1, as1, ad1, b1, W2, as2, ad2, b2,
           W3, as3, ad3, b3, Wl, bl, Wc, bc):
    del batch
    x_p = jnp.pad(x, ((0, NP - N), (0, 0)))

    h = _tc_first(x_p, W1, _att_pair(as1, ad1))
    p, _ = _sc_layer_fn()(edge_index, *h)

    h = _tc_mid(p, b1.reshape(1, D), W2, _att_pair(as2, ad2))
    p, _ = _sc_layer_fn()(edge_index, *h)

    h = _tc_mid(p, b2.reshape(1, D), W3, _att_pair(as3, ad3))
    p, _ = _sc_layer_fn()(edge_index, *h)

    out = _tc_final(p, b3.reshape(1, D), Wl, bl.reshape(1, D), Wc, bc.reshape(1, 3))
    return out[:N]


# speedup vs baseline: 12.3010x; 1.2364x over previous
import functools

import jax
import jax.numpy as jnp
from jax import lax
from jax.experimental import pallas as pl
from jax.experimental.pallas import tpu as pltpu
from jax.experimental.pallas import tpu_sc as plsc

N = 10000
NP = 10240
E = 160000
D = 512
BLK = 1024
NC, NS = 2, 16
RPT = NP // NS
B = 128
NBT = E // B
NB_HALF = NBT // (NC * NS)
LEFT = NBT // NC - NS * NB_HALF
AC_W = 128

_f32 = jnp.float32


def _dot(a, b):
    return jnp.dot(a, b, preferred_element_type=_f32)


def _tc_first_body(x_ref, w_ref, a_ref,
                   h0, h1, h2, h3, ac_ref):
    h = _dot(x_ref[...], w_ref[...])
    ac_ref[...] = jnp.pad(_dot(h, a_ref[...]), ((0, 0), (0, AC_W - 32)))
    for k, hr in enumerate((h0, h1, h2, h3)):
        hr[...] = h[:, 128 * k:128 * (k + 1)]


def _tc_mid_body(p_ref, b_ref, w_ref, a_ref,
                 h0, h1, h2, h3, ac_ref):
    p = p_ref[...]
    hin = jnp.concatenate([p[0, k] + p[1, k] for k in range(4)], axis=-1)
    hin = jnp.maximum(hin + b_ref[...], 0.0)
    h = _dot(hin, w_ref[...])
    ac_ref[...] = jnp.pad(_dot(h, a_ref[...]), ((0, 0), (0, AC_W - 32)))
    for k, hr in enumerate((h0, h1, h2, h3)):
        hr[...] = h[:, 128 * k:128 * (k + 1)]


def _tc_final_body(p_ref, b3_ref, wl_ref, bl_ref, wc_ref, bc_ref, out_ref):
    p = p_ref[...]
    hin = jnp.concatenate([p[0, k] + p[1, k] for k in range(4)], axis=-1)
    hin = jnp.maximum(hin + b3_ref[...], 0.0)
    hl = jnp.maximum(_dot(hin, wl_ref[...]) + bl_ref[...], 0.0)
    out_ref[...] = _dot(hl, wc_ref[...]) + bc_ref[...]


def _const_spec(shape):
    return pl.BlockSpec(shape, lambda b: (0,) * len(shape))


def _row_spec(shape):
    return pl.BlockSpec(shape, lambda b: (b,) + (0,) * (len(shape) - 1))


_H_OUT = (
    [jax.ShapeDtypeStruct((NP, 128), _f32) for _ in range(4)]
    + [jax.ShapeDtypeStruct((NP, AC_W), _f32)]
)
_H_OUT_SPECS = (
    [_row_spec((BLK, 128)) for _ in range(4)]
    + [_row_spec((BLK, AC_W))]
)


@jax.jit
def _tc_first(x_p, w, a):
    return pl.pallas_call(
        _tc_first_body,
        grid=(NP // BLK,),
        in_specs=[
            _row_spec((BLK, 128)),
            _const_spec((128, D)),
            _const_spec((D, 32)),
        ],
        out_specs=_H_OUT_SPECS,
        out_shape=_H_OUT,
    )(x_p, w, a)


@jax.jit
def _tc_mid(p, bprev, w, a):
    return pl.pallas_call(
        _tc_mid_body,
        grid=(NP // BLK,),
        in_specs=[
            pl.BlockSpec((2, 4, BLK, 128), lambda b: (0, 0, b, 0)),
            _const_spec((1, D)),
            _const_spec((D, D)),
            _const_spec((D, 32)),
        ],
        out_specs=_H_OUT_SPECS,
        out_shape=_H_OUT,
    )(p, bprev, w, a)


@jax.jit
def _tc_final(p, b3, wl, bl, wc, bc):
    return pl.pallas_call(
        _tc_final_body,
        grid=(NP // BLK,),
        in_specs=[
            pl.BlockSpec((2, 4, BLK, 128), lambda b: (0, 0, b, 0)),
            _const_spec((1, D)),
            _const_spec((D, D)),
            _const_spec((1, D)),
            _const_spec((D, 3)),
            _const_spec((1, 3)),
        ],
        out_specs=_row_spec((BLK, 3)),
        out_shape=jax.ShapeDtypeStruct((NP, 3), _f32),
    )(p, b3, wl, bl, wc, bc)


def _sc_body(ei, h0, h1, h2, h3, ac,
             p_out, a_out,
             acc_sh,
             eidx,
             exw, hrow, alph_v):
    c = lax.axis_index("c")
    s = lax.axis_index("s")
    rows0 = s * RPT

    zero16 = jnp.zeros((16,), _f32)

    def _zero_rows(ref):
        def zb(i, _):
            for v in range(8):
                ref[i, pl.ds(16 * v, 16)] = zero16
            return 0
        lax.fori_loop(0, B, zb, 0)

    def _zero_acc():
        for j in range(RPT // B):
            pltpu.sync_copy(hrow, acc_sh.at[pl.ds(rows0 + j * B, B)])

    _zero_rows(hrow)
    _zero_acc()
    plsc.subcore_barrier()

    HALF = NBT // NC
    base_own = c * HALF + s * NB_HALF
    base_mir = (1 - c) * HALF + s * NB_HALF
    nb = NB_HALF + jnp.where(s == NS - 1, LEFT, 0)

    def _eoff(batch_idx):
        return pl.multiple_of(batch_idx * B, B)

    def _gather_ex():
        pltpu.sync_copy(ac.at[eidx.at[0]], hrow)
        pltpu.sync_copy(ac.at[eidx.at[1]], exw)

        def body(i, _):
            e = hrow[i, pl.ds(0, 16)] + exw[i, pl.ds(16, 16)]
            e = jnp.maximum(e, 0.2 * e)
            exw[i, pl.ds(0, 16)] = jnp.exp(e)
            exw[i, pl.ds(16, 16)] = zero16
            return 0

        lax.fori_loop(0, B, body, 0)

    def _a1(bi, store):
        off = _eoff(bi)
        pltpu.sync_copy(ei.at[:, pl.ds(off, B)], eidx)
        _gather_ex()
        if store:
            def st(i8, _2):
                for v in range(8):
                    alph_v[i8, pl.ds(16 * v, 16)] = \
                        exw[8 * i8 + v, pl.ds(0, 16)]
                return 0

            lax.fori_loop(0, B // 8, st, 0)
            pltpu.sync_copy(alph_v, a_out.at[bi])
        pltpu.sync_copy(exw, acc_sh.at[eidx.at[1]], add=True)
        return 0

    lax.fori_loop(base_own, base_own + nb,
                  lambda bi, x: _a1(bi, True), 0)
    lax.fori_loop(base_mir, base_mir + nb,
                  lambda bi, x: _a1(bi, False), 0)
    plsc.subcore_barrier()

    def _a2(bi, _):
        off = _eoff(bi)
        pltpu.sync_copy(ei.at[:, pl.ds(off, B)], eidx)
        pltpu.sync_copy(acc_sh.at[eidx.at[1]], hrow)
        pltpu.sync_copy(a_out.at[bi], alph_v)

        def body(i8, _2):
            for v in range(8):
                alph_v[i8, pl.ds(16 * v, 16)] = (
                    alph_v[i8, pl.ds(16 * v, 16)]
                    / (hrow[8 * i8 + v, pl.ds(0, 16)] + 1e-16))
            return 0

        lax.fori_loop(0, B // 8, body, 0)
        pltpu.sync_copy(alph_v, a_out.at[bi])
        return 0

    lax.fori_loop(base_own, base_own + nb, _a2, 0)
    plsc.subcore_barrier()

    _zero_rows(hrow)
    _zero_acc()
    plsc.subcore_barrier()

    for k, hk in enumerate((h0, h1, h2, h3)):
        def _b(bi, _, k=k, hk=hk):
            off = _eoff(bi)
            pltpu.sync_copy(ei.at[:, pl.ds(off, B)], eidx)
            pltpu.sync_copy(hk.at[eidx.at[0]], hrow)
            pltpu.sync_copy(a_out.at[bi], alph_v)

            def body(i8, _2):
                for v in range(8):
                    arow = alph_v[i8, pl.ds(16 * v, 16)]
                    a0 = arow[2 * k]
                    a1 = arow[2 * k + 1]
                    r = 8 * i8 + v
                    for u in range(8):
                        au = a0 if u < 4 else a1
                        hrow[r, pl.ds(16 * u, 16)] = \
                            hrow[r, pl.ds(16 * u, 16)] * au
                return 0

            lax.fori_loop(0, B // 8, body, 0)
            pltpu.sync_copy(hrow, acc_sh.at[eidx.at[1]], add=True)
            return 0

        lax.fori_loop(base_own, base_own + nb, _b, 0)
        plsc.subcore_barrier()

        @pl.when(c == 0)
        def _():
            pltpu.sync_copy(acc_sh.at[pl.ds(rows0, RPT)],
                            p_out.at[0, k, pl.ds(rows0, RPT)])

        @pl.when(c == 1)
        def _():
            pltpu.sync_copy(acc_sh.at[pl.ds(rows0, RPT)],
                            p_out.at[1, k, pl.ds(rows0, RPT)])

        if k < 3:
            _zero_rows(hrow)
            _zero_acc()
        plsc.subcore_barrier()


@functools.cache
def _sc_layer_fn():
    mesh = plsc.VectorSubcoreMesh(
        core_axis_name="c", subcore_axis_name="s",
        num_cores=NC, num_subcores=NS)
    return pl.kernel(
    _sc_body,
        out_type=[jax.ShapeDtypeStruct((2, 4, NP, 128), _f32),
                  jax.ShapeDtypeStruct((NBT, B // 8, 128), _f32)],
        mesh=mesh,
        scratch_types=[
            pltpu.VMEM_SHARED((NP, 128), _f32),
            pltpu.VMEM((2, B), jnp.int32),
            pltpu.VMEM((B, 128), _f32),
            pltpu.VMEM((B, 128), _f32),
            pltpu.VMEM((B // 8, 128), _f32),
        ],
    )


def _att_mat(a):
    h, cdim = a.shape
    if h == 1:
        m = jnp.tile(a.reshape(-1, 1), (1, 8))
    else:
        m = (a[:, :, None] * jnp.eye(h, dtype=a.dtype)[:, None, :]).reshape(h * cdim, h)
    return jnp.concatenate([m, m], axis=1)


def _att_pair(a_s, a_d):
    return jnp.concatenate([_att_mat(a_s), _att_mat(a_d)], axis=1)


def kernel(x, edge_index, batch, W1, as1, ad1, b1, W2, as2, ad2, b2,
           W3, as3, ad3, b3, Wl, bl, Wc, bc):
    del batch
    x_p = jnp.pad(x, ((0, NP - N), (0, 0)))

    h = _tc_first(x_p, W1, _att_pair(as1, ad1))
    p, _ = _sc_layer_fn()(edge_index, *h)

    h = _tc_mid(p, b1.reshape(1, D), W2, _att_pair(as2, ad2))
    p, _ = _sc_layer_fn()(edge_index, *h)

    h = _tc_mid(p, b2.reshape(1, D), W3, _att_pair(as3, ad3))
    p, _ = _sc_layer_fn()(edge_index, *h)

    out = _tc_final(p, b3.reshape(1, D), Wl, bl.reshape(1, D), Wc, bc.reshape(1, 3))
    return out[:N]
